# baseline (speedup 1.0000x reference)
import functools

import numpy as np
import jax
import jax.numpy as jnp
from jax.experimental import pallas as pl
from jax.experimental.pallas import tpu as pltpu

_LEVELS = np.array([8, 8, 8, 5, 5, 5], dtype=np.int64)
_D = 6
_EPS = 1e-3

_HALF_L = (_LEVELS.astype(np.float64) - 1.0) * (1.0 - _EPS) / 2.0
_OFFSET = np.where(_LEVELS % 2 == 0, 0.5, 0.0)
_SHIFT = np.arctanh(_OFFSET / _HALF_L)
_HALF_W = (_LEVELS // 2).astype(np.float64)
_BASIS = np.concatenate([[1], np.cumprod(_LEVELS[:-1])]).astype(np.float64)


def _fsq_body(x_ref, w_in_ref, b_in_ref, w_out_ref, b_out_ref, consts_ref,
              q_x_ref, idx_ref):
    half_l = consts_ref[0, :]
    offset = consts_ref[1, :]
    shift = consts_ref[2, :]
    half_w = consts_ref[3, :]
    inv_half_w = consts_ref[4, :]
    basis = consts_ref[5, :]

    z = jax.lax.dot_general(
        x_ref[...], w_in_ref[...], (((1,), (0,)), ((), ())),
        preferred_element_type=jnp.float32)
    z = z + b_in_ref[...]
    bounded = jnp.tanh(z + shift) * half_l - offset
    q = jnp.round(bounded)
    codes = q * inv_half_w
    idx_ref[...] = jnp.sum((q + half_w) * basis, axis=-1).astype(jnp.int32)
    q_x = jax.lax.dot_general(
        codes, w_out_ref[...], (((1,), (0,)), ((), ())),
        preferred_element_type=jnp.float32)
    q_x_ref[...] = q_x + b_out_ref[...]


@functools.partial(jax.jit, static_argnames=())
def _fsq(x, W_in, b_in, W_out, b_out):
    B, T, C = x.shape
    n_tok = B * T
    x2 = x.reshape(n_tok, C)
    blk = 1024
    grid = (n_tok // blk,)
    consts = jnp.asarray(
        np.stack([_HALF_L, _OFFSET, _SHIFT, _HALF_W, 1.0 / _HALF_W, _BASIS]),
        dtype=jnp.float32)

    q_x, idx = pl.pallas_call(
        _fsq_body,
        grid=grid,
        in_specs=[
            pl.BlockSpec((blk, C), lambda i: (i, 0)),
            pl.BlockSpec((C, _D), lambda i: (0, 0)),
            pl.BlockSpec((_D,), lambda i: (0,)),
            pl.BlockSpec((_D, C), lambda i: (0, 0)),
            pl.BlockSpec((C,), lambda i: (0,)),
            pl.BlockSpec((6, _D), lambda i: (0, 0)),
        ],
        out_specs=[
            pl.BlockSpec((blk, C), lambda i: (i, 0)),
            pl.BlockSpec((blk,), lambda i: (i,)),
        ],
        out_shape=[
            jax.ShapeDtypeStruct((n_tok, C), jnp.float32),
            jax.ShapeDtypeStruct((n_tok,), jnp.int32),
        ],
        compiler_params=pltpu.CompilerParams(
            dimension_semantics=("parallel",)),
    )(x2, W_in, b_in, W_out, b_out, consts)

    return q_x.reshape(B, T, C), idx.reshape(B, T)


def kernel(x, W_in, b_in, W_out, b_out):
    return _fsq(x, W_in, b_in, W_out, b_out)


# speedup vs baseline: 1.5592x; 1.5592x over previous
import numpy as np
import jax
import jax.numpy as jnp
from jax.experimental import pallas as pl
from jax.experimental.pallas import tpu as pltpu

_LEVELS = np.array([8, 8, 8, 5, 5, 5], dtype=np.int64)
_D = 6
_EPS = 1e-3

_HALF_L = (_LEVELS.astype(np.float64) - 1.0) * (1.0 - _EPS) / 2.0
_OFFSET = np.where(_LEVELS % 2 == 0, 0.5, 0.0)
_SHIFT = np.arctanh(_OFFSET / _HALF_L)
_HALF_W = (_LEVELS // 2).astype(np.float64)
_BASIS = np.concatenate([[1], np.cumprod(_LEVELS[:-1])]).astype(np.float64)
_IDX_OFFSET = float(np.sum(_HALF_W * _BASIS))

_BT = 1152
_NBUF = 4


def _fsq_body(x_hbm, w_in_t_ref, w_out_ref, b_out_ref, consts_ref,
              q_x_hbm, idx_ref, x_buf, q_buf, in_sems, out_sems):
    n_tok = x_hbm.shape[0]
    nc = n_tok // _BT

    def in_copy(c, s):
        return pltpu.make_async_copy(
            x_hbm.at[pl.ds(c * _BT, _BT)], x_buf.at[s], in_sems.at[s])

    def out_copy(c, s):
        return pltpu.make_async_copy(
            q_buf.at[s], q_x_hbm.at[pl.ds(c * _BT, _BT)], out_sems.at[s])

    for s in range(min(_NBUF, nc)):
        in_copy(s, s).start()

    half_l = consts_ref[:, 0:1]
    offset = consts_ref[:, 1:2]
    shift = consts_ref[:, 2:3]
    inv_half_w = consts_ref[:, 3:4]
    basis = consts_ref[:, 4:5]
    b_in = consts_ref[:, 5:6]

    def step(i, carry):
        s = jax.lax.rem(i, _NBUF)
        in_copy(i, s).wait()
        z_t = jax.lax.dot_general(
            w_in_t_ref[...], x_buf[s], (((1,), (1,)), ((), ())),
            preferred_element_type=jnp.float32) + b_in

        @pl.when(i + _NBUF < nc)
        def _prefetch():
            in_copy(i + _NBUF, s).start()

        bounded = jnp.tanh(z_t + shift) * half_l - offset
        q = jnp.round(bounded)
        codes_t = q * inv_half_w
        idx = jnp.sum(q * basis, axis=0) + _IDX_OFFSET
        idx_ref[i] = idx.astype(jnp.int32).reshape(1, _BT)
        q_x = jax.lax.dot_general(
            codes_t, w_out_ref[...], (((0,), (0,)), ((), ())),
            preferred_element_type=jnp.float32)

        @pl.when(i >= _NBUF)
        def _drain():
            out_copy(i - _NBUF, s).wait()

        q_buf[s] = q_x + b_out_ref[...]
        out_copy(i, s).start()
        return carry

    jax.lax.fori_loop(0, nc, step, 0)

    for c in range(max(0, nc - _NBUF), nc):
        out_copy(c, c % _NBUF).wait()


@jax.jit
def _fsq(x, W_in, b_in, W_out, b_out):
    B, T, C = x.shape
    n_tok = B * T
    nc = n_tok // _BT
    x2 = x.reshape(n_tok, C)
    consts = jnp.asarray(
        np.stack([_HALF_L, _OFFSET, _SHIFT, 1.0 / _HALF_W, _BASIS,
                  np.zeros(_D)], axis=1),
        dtype=jnp.float32)
    consts = consts.at[:, 5].set(b_in)
    w_in_t = W_in.T

    q_x, idx = pl.pallas_call(
        _fsq_body,
        in_specs=[
            pl.BlockSpec(memory_space=pl.ANY),
            pl.BlockSpec(memory_space=pltpu.VMEM),
            pl.BlockSpec(memory_space=pltpu.VMEM),
            pl.BlockSpec(memory_space=pltpu.VMEM),
            pl.BlockSpec(memory_space=pltpu.VMEM),
        ],
        out_specs=[
            pl.BlockSpec(memory_space=pl.ANY),
            pl.BlockSpec(memory_space=pltpu.VMEM),
        ],
        out_shape=[
            jax.ShapeDtypeStruct((n_tok, C), jnp.float32),
            jax.ShapeDtypeStruct((nc, 1, _BT), jnp.int32),
        ],
        scratch_shapes=[
            pltpu.VMEM((_NBUF, _BT, C), jnp.float32),
            pltpu.VMEM((_NBUF, _BT, C), jnp.float32),
            pltpu.SemaphoreType.DMA((_NBUF,)),
            pltpu.SemaphoreType.DMA((_NBUF,)),
        ],
    )(x2, w_in_t, W_out, b_out.reshape(1, C), consts)

    return q_x.reshape(B, T, C), idx.reshape(B, T)


def kernel(x, W_in, b_in, W_out, b_out):
    return _fsq(x, W_in, b_in, W_out, b_out)


# speedup vs baseline: 1.6299x; 1.0453x over previous
import numpy as np
import jax
import jax.numpy as jnp
from jax.experimental import pallas as pl
from jax.experimental.pallas import tpu as pltpu

_LEVELS = np.array([8, 8, 8, 5, 5, 5], dtype=np.int64)
_D = 6
_EPS = 1e-3

_HALF_L = (_LEVELS.astype(np.float64) - 1.0) * (1.0 - _EPS) / 2.0
_OFFSET = np.where(_LEVELS % 2 == 0, 0.5, 0.0)
_SHIFT = np.arctanh(_OFFSET / _HALF_L)
_HALF_W = (_LEVELS // 2).astype(np.float64)
_BASIS = np.concatenate([[1], np.cumprod(_LEVELS[:-1])]).astype(np.float64)
_IDX_OFFSET = float(np.sum(_HALF_W * _BASIS))

_BT = 2304
_NBUF = 4


def _fsq_body(x_hbm, w_in_t_ref, w_out_ref, b_out_ref, consts_ref,
              q_x_hbm, idx_ref, x_buf, q_buf, in_sems, out_sems):
    n_tok = x_hbm.shape[0]
    nc = n_tok // _BT

    def in_copy(c, s):
        return pltpu.make_async_copy(
            x_hbm.at[pl.ds(c * _BT, _BT)], x_buf.at[s], in_sems.at[s])

    def out_copy(c, s):
        return pltpu.make_async_copy(
            q_buf.at[s], q_x_hbm.at[pl.ds(c * _BT, _BT)], out_sems.at[s])

    for s in range(min(_NBUF, nc)):
        in_copy(s, s).start()

    half_l = consts_ref[:, 0:1]
    offset = consts_ref[:, 1:2]
    shift = consts_ref[:, 2:3]
    inv_half_w = consts_ref[:, 3:4]
    basis = consts_ref[:, 4:5]
    b_in = consts_ref[:, 5:6]

    def step(i, carry):
        s = jax.lax.rem(i, _NBUF)
        in_copy(i, s).wait()
        z_t = jax.lax.dot_general(
            w_in_t_ref[...], x_buf[s], (((1,), (1,)), ((), ())),
            preferred_element_type=jnp.float32) + b_in

        @pl.when(i + _NBUF < nc)
        def _prefetch():
            in_copy(i + _NBUF, s).start()

        bounded = jnp.tanh(z_t + shift) * half_l - offset
        q = jnp.round(bounded)
        codes_t = q * inv_half_w
        idx = jnp.sum(q * basis, axis=0) + _IDX_OFFSET
        idx_ref[i] = idx.astype(jnp.int32).reshape(1, _BT)
        q_x = jax.lax.dot_general(
            codes_t, w_out_ref[...], (((0,), (0,)), ((), ())),
            preferred_element_type=jnp.float32)

        @pl.when(i >= _NBUF)
        def _drain():
            out_copy(i - _NBUF, s).wait()

        q_buf[s] = q_x + b_out_ref[...]
        out_copy(i, s).start()
        return carry

    jax.lax.fori_loop(0, nc, step, 0)

    for c in range(max(0, nc - _NBUF), nc):
        out_copy(c, c % _NBUF).wait()


@jax.jit
def _fsq(x, W_in, b_in, W_out, b_out):
    B, T, C = x.shape
    n_tok = B * T
    nc = n_tok // _BT
    x2 = x.reshape(n_tok, C)
    consts = jnp.asarray(
        np.stack([_HALF_L, _OFFSET, _SHIFT, 1.0 / _HALF_W, _BASIS,
                  np.zeros(_D)], axis=1),
        dtype=jnp.float32)
    consts = consts.at[:, 5].set(b_in)
    w_in_t = W_in.T

    q_x, idx = pl.pallas_call(
        _fsq_body,
        in_specs=[
            pl.BlockSpec(memory_space=pl.ANY),
            pl.BlockSpec(memory_space=pltpu.VMEM),
            pl.BlockSpec(memory_space=pltpu.VMEM),
            pl.BlockSpec(memory_space=pltpu.VMEM),
            pl.BlockSpec(memory_space=pltpu.VMEM),
        ],
        out_specs=[
            pl.BlockSpec(memory_space=pl.ANY),
            pl.BlockSpec(memory_space=pltpu.VMEM),
        ],
        out_shape=[
            jax.ShapeDtypeStruct((n_tok, C), jnp.float32),
            jax.ShapeDtypeStruct((nc, 1, _BT), jnp.int32),
        ],
        scratch_shapes=[
            pltpu.VMEM((_NBUF, _BT, C), jnp.float32),
            pltpu.VMEM((_NBUF, _BT, C), jnp.float32),
            pltpu.SemaphoreType.DMA((_NBUF,)),
            pltpu.SemaphoreType.DMA((_NBUF,)),
        ],
    )(x2, w_in_t, W_out, b_out.reshape(1, C), consts)

    return q_x.reshape(B, T, C), idx.reshape(B, T)


def kernel(x, W_in, b_in, W_out, b_out):
    return _fsq(x, W_in, b_in, W_out, b_out)
